# baseline (speedup 1.0000x reference)
import functools

import jax
import jax.numpy as jnp
from jax import lax
from jax.experimental import pallas as pl
from jax.experimental.pallas import tpu as pltpu
from jax.experimental.pallas import tpu_sc as plsc

B = 16384
EMB = 32
WIDE = 128
NC = 2
NS = 16
NW = NC * NS
CHUNK = 128
ROWS = B // CHUNK
RPW = ROWS // NW
BPW = RPW * CHUNK
NSTEP = 2 * RPW


def _tr_body(in_ref, out_ref):
    out_ref[...] = in_ref[...].T


def _transpose(x_t, blk):
    n = x_t.shape[1]
    grid = (pl.cdiv(n, blk),)
    return pl.pallas_call(
        _tr_body,
        grid=grid,
        in_specs=[pl.BlockSpec((EMB, blk), lambda i: (0, i))],
        out_specs=pl.BlockSpec((blk, EMB), lambda i: (i, 0)),
        out_shape=jax.ShapeDtypeStruct((n, EMB), jnp.float32),
    )(x_t)


def _gather_body(ur_hbm, mr_hbm, uq_hbm, mq_hbm, utab_w, mtab_w,
                 ue_out, me_out,
                 ur_v, mr_v, uq_v, mq_v, wide_a, wide_b, sel_a, sel_b,
                 gsem0, gsem1, wsem):
    wid = lax.axis_index("s") * NC + lax.axis_index("c")
    r0 = wid * RPW
    b0 = wid * BPW
    pltpu.sync_copy(ur_hbm.at[pl.ds(r0, RPW)], ur_v)
    pltpu.sync_copy(mr_hbm.at[pl.ds(r0, RPW)], mr_v)
    pltpu.sync_copy(uq_hbm.at[pl.ds(b0, BPW)], uq_v)
    pltpu.sync_copy(mq_hbm.at[pl.ds(b0, BPW)], mq_v)
    gsems = (gsem0, gsem1)
    wides = (wide_a, wide_b)

    def step_src(s):
        if s < RPW:
            return utab_w.at[ur_v.at[s]]
        return mtab_w.at[mr_v.at[s - RPW]]

    sels = (sel_a, sel_b)

    def select(s):
        j = s % RPW
        q_v = uq_v if s < RPW else mq_v
        sel_v = sels[s % 2]
        wide2d = wides[s % 2]

        @pl.loop(0, CHUNK // 16)
        def _(g):
            q16 = q_v[pl.ds(j * CHUNK + g * 16, 16)] * EMB
            for l in range(16):
                row = g * 16 + l
                qoff = q16[l]
                sel_v[row, pl.ds(0, 16)] = wide2d[row, pl.ds(qoff, 16)]
                sel_v[row, pl.ds(16, 16)] = wide2d[row, pl.ds(qoff + 16, 16)]

    def step_out(s):
        j = s % RPW
        out = ue_out if s < RPW else me_out
        return out.at[pl.ds(b0 + j * CHUNK, CHUNK)]

    gathers = [None] * NSTEP
    writes = [None] * NSTEP
    gathers[0] = pltpu.async_copy(step_src(0), wides[0], gsems[0])
    for s in range(NSTEP):
        if s + 1 < NSTEP:
            gathers[s + 1] = pltpu.async_copy(step_src(s + 1),
                                              wides[(s + 1) % 2],
                                              gsems[(s + 1) % 2])
        gathers[s].wait()
        if s >= 2:
            writes[s - 2].wait()
        select(s)
        writes[s] = pltpu.async_copy(sels[s % 2], step_out(s), wsem)
    writes[NSTEP - 2].wait()
    writes[NSTEP - 1].wait()


@functools.partial(
    pl.kernel,
    mesh=plsc.VectorSubcoreMesh(core_axis_name="c", subcore_axis_name="s",
                                num_cores=NC),
    out_type=[
        jax.ShapeDtypeStruct((B, EMB), jnp.float32),
        jax.ShapeDtypeStruct((B, EMB), jnp.float32),
    ],
    scratch_types=[
        pltpu.VMEM((RPW, CHUNK), jnp.int32),
        pltpu.VMEM((RPW, CHUNK), jnp.int32),
        pltpu.VMEM((BPW,), jnp.int32),
        pltpu.VMEM((BPW,), jnp.int32),
        pltpu.VMEM((CHUNK, WIDE), jnp.float32),
        pltpu.VMEM((CHUNK, WIDE), jnp.float32),
        pltpu.VMEM((CHUNK, EMB), jnp.float32),
        pltpu.VMEM((CHUNK, EMB), jnp.float32),
        pltpu.SemaphoreType.DMA,
        pltpu.SemaphoreType.DMA,
        pltpu.SemaphoreType.DMA,
    ],
)
def _gather(*args):
    _gather_body(*args)


def _mlp_body(ue_ref, me_ref, w1a_ref, w1b_ref, b1_ref, w2_ref, b2_ref,
              w3_ref, b3_ref, out_ref):
    x = jnp.dot(ue_ref[...], w1a_ref[...], preferred_element_type=jnp.float32)
    x = x + jnp.dot(me_ref[...], w1b_ref[...],
                    preferred_element_type=jnp.float32)
    x = jnp.maximum(x + b1_ref[...], 0.0)
    x = jnp.maximum(
        jnp.dot(x, w2_ref[...], preferred_element_type=jnp.float32)
        + b2_ref[...], 0.0)
    out_ref[...] = (jnp.dot(x, w3_ref[...], preferred_element_type=jnp.float32)
                    + b3_ref[...])


def _mlp(ue, me, W1a, W1b, b1, W2, b2, W3, b3):
    BB = 2048
    grid = (B // BB,)
    full = lambda shape: pl.BlockSpec(shape, lambda i: (0, 0))
    return pl.pallas_call(
        _mlp_body,
        grid=grid,
        in_specs=[
            pl.BlockSpec((BB, EMB), lambda i: (i, 0)),
            pl.BlockSpec((BB, EMB), lambda i: (i, 0)),
            full((EMB, 64)),
            full((EMB, 64)),
            full((1, 64)),
            full((64, 32)),
            full((1, 32)),
            full((32, 1)),
            full((1, 1)),
        ],
        out_specs=pl.BlockSpec((BB, 1), lambda i: (i, 0)),
        out_shape=jax.ShapeDtypeStruct((B, 1), jnp.float32),
    )(ue, me, W1a, W1b, b1, W2, b2, W3, b3)


def kernel(user, movie, user_table, movie_table, W1, b1, W2, b2, W3, b3):
    user = user.astype(jnp.int32)
    movie = movie.astype(jnp.int32)
    ur = (user >> 2).reshape(ROWS, CHUNK)
    mr = (movie >> 2).reshape(ROWS, CHUNK)
    uq = user & 3
    mq = movie & 3
    utab_w = _transpose(user_table.T, 8192).reshape(-1, WIDE)
    mtab_w = _transpose(movie_table.T, 4096).reshape(-1, WIDE)
    ue, me = _gather(ur, mr, uq, mq, utab_w, mtab_w)
    return _mlp(ue, me, W1[:EMB], W1[EMB:], b1.reshape(1, 64),
                W2, b2.reshape(1, 32), W3, b3.reshape(1, 1))


# speedup vs baseline: 2.0911x; 2.0911x over previous
import functools

import jax
import jax.numpy as jnp
from jax import lax
from jax.experimental import pallas as pl
from jax.experimental.pallas import tpu as pltpu
from jax.experimental.pallas import tpu_sc as plsc

B = 16384
EMB = 32
WIDE = 128
NC = 2
NS = 16
NW = NC * NS
CHUNK = 128
ROWS = B // CHUNK
RPW = ROWS // NW
BPW = RPW * CHUNK
NSTEP = 2 * RPW


def _tr_body(in_ref, out_ref):
    blk = in_ref.shape[1]
    xt = in_ref[...].T
    for c in range(4):
        out_ref[:, pl.ds(c * EMB, EMB)] = xt[c * (blk // 4):(c + 1) * (blk // 4), :]


def _transpose(x_t, blk):
    n = x_t.shape[1]
    nb = pl.cdiv(n, blk)
    return pl.pallas_call(
        _tr_body,
        grid=(nb,),
        in_specs=[pl.BlockSpec((EMB, blk), lambda i: (0, i))],
        out_specs=pl.BlockSpec((blk // 4, WIDE), lambda i: (i, 0)),
        out_shape=jax.ShapeDtypeStruct((nb * (blk // 4), WIDE), jnp.float32),
    )(x_t)


def _gather_body(ur_hbm, mr_hbm, uq_hbm, mq_hbm, utab_w, mtab_w,
                 ue_out, me_out,
                 ur_v, mr_v, uq_v, mq_v, wide_a, wide_b, sel_a, sel_b,
                 gsem0, gsem1, wsem):
    wid = lax.axis_index("s") * NC + lax.axis_index("c")
    r0 = wid * RPW
    b0 = wid * BPW
    pltpu.sync_copy(ur_hbm.at[pl.ds(r0, RPW)], ur_v)
    pltpu.sync_copy(mr_hbm.at[pl.ds(r0, RPW)], mr_v)
    pltpu.sync_copy(uq_hbm.at[pl.ds(b0, BPW)], uq_v)
    pltpu.sync_copy(mq_hbm.at[pl.ds(b0, BPW)], mq_v)
    gsems = (gsem0, gsem1)
    wides = (wide_a, wide_b)

    def step_src(s):
        if s < RPW:
            return utab_w.at[ur_v.at[s]]
        return mtab_w.at[mr_v.at[s - RPW]]

    sels = (sel_a, sel_b)

    def select(s):
        j = s % RPW
        q_v = uq_v if s < RPW else mq_v
        sel_v = sels[s % 2]
        wide2d = wides[s % 2]

        @pl.loop(0, CHUNK // 16)
        def _(g):
            q16 = q_v[pl.ds(j * CHUNK + g * 16, 16)] * EMB
            for l in range(16):
                row = g * 16 + l
                qoff = q16[l]
                sel_v[row, pl.ds(0, 16)] = wide2d[row, pl.ds(qoff, 16)]
                sel_v[row, pl.ds(16, 16)] = wide2d[row, pl.ds(qoff + 16, 16)]

    def step_out(s):
        j = s % RPW
        out = ue_out if s < RPW else me_out
        return out.at[pl.ds(b0 + j * CHUNK, CHUNK)]

    gathers = [None] * NSTEP
    writes = [None] * NSTEP
    gathers[0] = pltpu.async_copy(step_src(0), wides[0], gsems[0])
    for s in range(NSTEP):
        if s + 1 < NSTEP:
            gathers[s + 1] = pltpu.async_copy(step_src(s + 1),
                                              wides[(s + 1) % 2],
                                              gsems[(s + 1) % 2])
        gathers[s].wait()
        if s >= 2:
            writes[s - 2].wait()
        select(s)
        writes[s] = pltpu.async_copy(sels[s % 2], step_out(s), wsem)
    writes[NSTEP - 2].wait()
    writes[NSTEP - 1].wait()


@functools.partial(
    pl.kernel,
    mesh=plsc.VectorSubcoreMesh(core_axis_name="c", subcore_axis_name="s",
                                num_cores=NC),
    out_type=[
        jax.ShapeDtypeStruct((B, EMB), jnp.float32),
        jax.ShapeDtypeStruct((B, EMB), jnp.float32),
    ],
    scratch_types=[
        pltpu.VMEM((RPW, CHUNK), jnp.int32),
        pltpu.VMEM((RPW, CHUNK), jnp.int32),
        pltpu.VMEM((BPW,), jnp.int32),
        pltpu.VMEM((BPW,), jnp.int32),
        pltpu.VMEM((CHUNK, WIDE), jnp.float32),
        pltpu.VMEM((CHUNK, WIDE), jnp.float32),
        pltpu.VMEM((CHUNK, EMB), jnp.float32),
        pltpu.VMEM((CHUNK, EMB), jnp.float32),
        pltpu.SemaphoreType.DMA,
        pltpu.SemaphoreType.DMA,
        pltpu.SemaphoreType.DMA,
    ],
)
def _gather(*args):
    _gather_body(*args)


def _mlp_body(ue_ref, me_ref, w1a_ref, w1b_ref, b1_ref, w2_ref, b2_ref,
              w3_ref, b3_ref, out_ref):
    x = jnp.dot(ue_ref[...], w1a_ref[...], preferred_element_type=jnp.float32)
    x = x + jnp.dot(me_ref[...], w1b_ref[...],
                    preferred_element_type=jnp.float32)
    x = jnp.maximum(x + b1_ref[...], 0.0)
    x = jnp.maximum(
        jnp.dot(x, w2_ref[...], preferred_element_type=jnp.float32)
        + b2_ref[...], 0.0)
    out_ref[...] = (jnp.dot(x, w3_ref[...], preferred_element_type=jnp.float32)
                    + b3_ref[...])


def _mlp(ue, me, W1a, W1b, b1, W2, b2, W3, b3):
    BB = 2048
    grid = (B // BB,)
    full = lambda shape: pl.BlockSpec(shape, lambda i: (0, 0))
    return pl.pallas_call(
        _mlp_body,
        grid=grid,
        in_specs=[
            pl.BlockSpec((BB, EMB), lambda i: (i, 0)),
            pl.BlockSpec((BB, EMB), lambda i: (i, 0)),
            full((EMB, 64)),
            full((EMB, 64)),
            full((1, 64)),
            full((64, 32)),
            full((1, 32)),
            full((32, 1)),
            full((1, 1)),
        ],
        out_specs=pl.BlockSpec((BB, 1), lambda i: (i, 0)),
        out_shape=jax.ShapeDtypeStruct((B, 1), jnp.float32),
    )(ue, me, W1a, W1b, b1, W2, b2, W3, b3)


def kernel(user, movie, user_table, movie_table, W1, b1, W2, b2, W3, b3):
    user = user.astype(jnp.int32)
    movie = movie.astype(jnp.int32)
    ur = ((user >> 13) * 2048 + (user & 2047)).reshape(ROWS, CHUNK)
    mr = ((movie >> 13) * 2048 + (movie & 2047)).reshape(ROWS, CHUNK)
    uq = (user >> 11) & 3
    mq = (movie >> 11) & 3
    utab_w = _transpose(user_table.T, 8192)
    mtab_w = _transpose(movie_table.T, 8192)
    ue, me = _gather(ur, mr, uq, mq, utab_w, mtab_w)
    return _mlp(ue, me, W1[:EMB], W1[EMB:], b1.reshape(1, 64),
                W2, b2.reshape(1, 32), W3, b3.reshape(1, 1))


# speedup vs baseline: 2.6076x; 1.2470x over previous
import functools

import jax
import jax.numpy as jnp
from jax import lax
from jax.experimental import pallas as pl
from jax.experimental.pallas import tpu as pltpu
from jax.experimental.pallas import tpu_sc as plsc

B = 16384
EMB = 32
WIDE = 128
NC = 2
NS = 16
NW = NC * NS
CHUNK = 128
ROWS = B // CHUNK
RPW = ROWS // NW
BPW = RPW * CHUNK
NSTEP = 2 * RPW


def _tr_body(in_ref, out_ref):
    blk = in_ref.shape[1]
    q = blk // 4
    x = in_ref[...]
    row = lax.broadcasted_iota(jnp.int32, (EMB, WIDE), 0)
    col = lax.broadcasted_iota(jnp.int32, (EMB, WIDE), 1)
    out = None
    for c in range(4):
        e_c = (col == row + c * EMB).astype(jnp.float32)
        d = lax.dot_general(x[:, c * q:(c + 1) * q], e_c,
                            (((0,), (0,)), ((), ())),
                            preferred_element_type=jnp.float32)
        out = d if out is None else out + d
    out_ref[...] = out


def _transpose(x_t, blk):
    n = x_t.shape[1]
    nb = pl.cdiv(n, blk)
    return pl.pallas_call(
        _tr_body,
        grid=(nb,),
        in_specs=[pl.BlockSpec((EMB, blk), lambda i: (0, i))],
        out_specs=pl.BlockSpec((blk // 4, WIDE), lambda i: (i, 0)),
        out_shape=jax.ShapeDtypeStruct((nb * (blk // 4), WIDE), jnp.float32),
    )(x_t)


def _gather_body(ur_hbm, mr_hbm, uq_hbm, mq_hbm, utab_w, mtab_w,
                 ue_out, me_out,
                 ur_v, mr_v, uq_v, mq_v, wide_a, wide_b, sel_a, sel_b,
                 gsem0, gsem1, wsem):
    wid = lax.axis_index("s") * NC + lax.axis_index("c")
    r0 = wid * RPW
    b0 = wid * BPW
    pltpu.sync_copy(ur_hbm.at[pl.ds(r0, RPW)], ur_v)
    pltpu.sync_copy(mr_hbm.at[pl.ds(r0, RPW)], mr_v)
    pltpu.sync_copy(uq_hbm.at[pl.ds(b0, BPW)], uq_v)
    pltpu.sync_copy(mq_hbm.at[pl.ds(b0, BPW)], mq_v)
    gsems = (gsem0, gsem1)
    wides = (wide_a, wide_b)

    def step_src(s):
        if s < RPW:
            return utab_w.at[ur_v.at[s]]
        return mtab_w.at[mr_v.at[s - RPW]]

    sels = (sel_a, sel_b)

    def select(s):
        j = s % RPW
        q_v = uq_v if s < RPW else mq_v
        sel_v = sels[s % 2]
        wide2d = wides[s % 2]

        @pl.loop(0, CHUNK // 16)
        def _(g):
            q16 = q_v[pl.ds(j * CHUNK + g * 16, 16)] * EMB
            for l in range(16):
                row = g * 16 + l
                qoff = q16[l]
                sel_v[row, pl.ds(0, 16)] = wide2d[row, pl.ds(qoff, 16)]
                sel_v[row, pl.ds(16, 16)] = wide2d[row, pl.ds(qoff + 16, 16)]

    def step_out(s):
        j = s % RPW
        out = ue_out if s < RPW else me_out
        return out.at[pl.ds(b0 + j * CHUNK, CHUNK)]

    gathers = [None] * NSTEP
    writes = [None] * NSTEP
    gathers[0] = pltpu.async_copy(step_src(0), wides[0], gsems[0])
    for s in range(NSTEP):
        if s + 1 < NSTEP:
            gathers[s + 1] = pltpu.async_copy(step_src(s + 1),
                                              wides[(s + 1) % 2],
                                              gsems[(s + 1) % 2])
        gathers[s].wait()
        if s >= 2:
            writes[s - 2].wait()
        select(s)
        writes[s] = pltpu.async_copy(sels[s % 2], step_out(s), wsem)
    writes[NSTEP - 2].wait()
    writes[NSTEP - 1].wait()


@functools.partial(
    pl.kernel,
    mesh=plsc.VectorSubcoreMesh(core_axis_name="c", subcore_axis_name="s",
                                num_cores=NC),
    out_type=[
        jax.ShapeDtypeStruct((B, EMB), jnp.float32),
        jax.ShapeDtypeStruct((B, EMB), jnp.float32),
    ],
    scratch_types=[
        pltpu.VMEM((RPW, CHUNK), jnp.int32),
        pltpu.VMEM((RPW, CHUNK), jnp.int32),
        pltpu.VMEM((BPW,), jnp.int32),
        pltpu.VMEM((BPW,), jnp.int32),
        pltpu.VMEM((CHUNK, WIDE), jnp.float32),
        pltpu.VMEM((CHUNK, WIDE), jnp.float32),
        pltpu.VMEM((CHUNK, EMB), jnp.float32),
        pltpu.VMEM((CHUNK, EMB), jnp.float32),
        pltpu.SemaphoreType.DMA,
        pltpu.SemaphoreType.DMA,
        pltpu.SemaphoreType.DMA,
    ],
)
def _gather(*args):
    _gather_body(*args)


def _mlp_body(ue_ref, me_ref, w1a_ref, w1b_ref, b1_ref, w2_ref, b2_ref,
              w3_ref, b3_ref, out_ref):
    x = jnp.dot(ue_ref[...], w1a_ref[...], preferred_element_type=jnp.float32)
    x = x + jnp.dot(me_ref[...], w1b_ref[...],
                    preferred_element_type=jnp.float32)
    x = jnp.maximum(x + b1_ref[...], 0.0)
    x = jnp.maximum(
        jnp.dot(x, w2_ref[...], preferred_element_type=jnp.float32)
        + b2_ref[...], 0.0)
    out_ref[...] = (jnp.dot(x, w3_ref[...], preferred_element_type=jnp.float32)
                    + b3_ref[...])


def _mlp(ue, me, W1a, W1b, b1, W2, b2, W3, b3):
    BB = 2048
    grid = (B // BB,)
    full = lambda shape: pl.BlockSpec(shape, lambda i: (0, 0))
    return pl.pallas_call(
        _mlp_body,
        grid=grid,
        in_specs=[
            pl.BlockSpec((BB, EMB), lambda i: (i, 0)),
            pl.BlockSpec((BB, EMB), lambda i: (i, 0)),
            full((EMB, 64)),
            full((EMB, 64)),
            full((1, 64)),
            full((64, 32)),
            full((1, 32)),
            full((32, 1)),
            full((1, 1)),
        ],
        out_specs=pl.BlockSpec((BB, 1), lambda i: (i, 0)),
        out_shape=jax.ShapeDtypeStruct((B, 1), jnp.float32),
    )(ue, me, W1a, W1b, b1, W2, b2, W3, b3)


def kernel(user, movie, user_table, movie_table, W1, b1, W2, b2, W3, b3):
    user = user.astype(jnp.int32)
    movie = movie.astype(jnp.int32)
    ur = ((user >> 13) * 2048 + (user & 2047)).reshape(ROWS, CHUNK)
    mr = ((movie >> 13) * 2048 + (movie & 2047)).reshape(ROWS, CHUNK)
    uq = (user >> 11) & 3
    mq = (movie >> 11) & 3
    utab_w = _transpose(user_table.T, 8192)
    mtab_w = _transpose(movie_table.T, 8192)
    ue, me = _gather(ur, mr, uq, mq, utab_w, mtab_w)
    return _mlp(ue, me, W1[:EMB], W1[EMB:], b1.reshape(1, 64),
                W2, b2.reshape(1, 32), W3, b3.reshape(1, 1))
